# baseline (speedup 1.0000x reference)
import jax
import jax.numpy as jnp
from jax import lax
from jax.experimental import pallas as pl
from jax.experimental.pallas import tpu as pltpu
from jax.experimental.pallas import tpu_sc as plsc

BATCH = 128
NF = 32768
NC = 2
NS = 16
NW = NC * NS
ROWS_PER_W = BATCH // NW
CHUNK = 8192
NCHUNK = NF // CHUNK
LANES = 16
VECS_PER_CHUNK = CHUNK // LANES

_F32_BIG = 3.4e38


def _closest_body(in_hbm, prev_hbm, out_hbm, buf, pv_buf, res_buf, sem):
    wid = lax.axis_index("s") * NC + lax.axis_index("c")

    pltpu.sync_copy(prev_hbm.at[wid], pv_buf)

    for r in range(ROWS_PER_W):
        row = wid * ROWS_PER_W + r
        pv = pv_buf[pl.ds(r * LANES, LANES)]

        mind = jnp.full((LANES,), _F32_BIG, jnp.float32)
        minv = jnp.zeros((LANES,), jnp.float32)

        for c in range(NCHUNK):
            pltpu.sync_copy(in_hbm.at[row, pl.ds(c * CHUNK, CHUNK)], buf)

            def body(i, carry):
                mind, minv = carry
                v = buf[pl.ds(i * LANES, LANES)]
                d = jnp.abs(v - pv)
                pred = d < mind
                return jnp.where(pred, d, mind), jnp.where(pred, v, minv)

            mind, minv = lax.fori_loop(0, VECS_PER_CHUNK, body, (mind, minv))

        _, vs = plsc.sort_key_val(mind, minv)
        res_buf[...] = vs
        pltpu.sync_copy(res_buf, out_hbm.at[row])


@jax.jit
def _closest(inp, prev_bc):
    mesh = plsc.VectorSubcoreMesh(core_axis_name="c", subcore_axis_name="s")
    f = pl.kernel(
        _closest_body,
        out_type=jax.ShapeDtypeStruct((BATCH, LANES), jnp.float32),
        mesh=mesh,
        compiler_params=pltpu.CompilerParams(needs_layout_passes=False),
        scratch_types=[
            pltpu.VMEM((CHUNK,), jnp.float32),
            pltpu.VMEM((ROWS_PER_W * LANES,), jnp.float32),
            pltpu.VMEM((LANES,), jnp.float32),
            pltpu.SemaphoreType.DMA,
        ],
    )
    return f(inp, prev_bc)


def kernel(input, prev_output):
    prev_bc = jnp.broadcast_to(
        prev_output.reshape(NW, ROWS_PER_W, 1), (NW, ROWS_PER_W, LANES)
    ).reshape(NW, ROWS_PER_W * LANES)
    out = _closest(input, prev_bc)
    return out[:, :1]


# speedup vs baseline: 1.9694x; 1.9694x over previous
import jax
import jax.numpy as jnp
from jax import lax
from jax.experimental import pallas as pl
from jax.experimental.pallas import tpu as pltpu
from jax.experimental.pallas import tpu_sc as plsc

BATCH = 128
NF = 32768
NC = 2
NS = 16
NW = NC * NS
ROWS_PER_W = BATCH // NW
CHUNK = 16384
NCHUNK = NF // CHUNK
LANES = 16
UNROLL = 8
ITERS = CHUNK // (LANES * UNROLL)

_F32_BIG = 3.4e38


def _closest_body(in_hbm, prev_hbm, out_hbm, buf0, buf1, pv_buf, res_buf,
                  sem0, sem1):
    wid = lax.axis_index("s") * NC + lax.axis_index("c")

    pltpu.sync_copy(prev_hbm.at[wid], pv_buf)

    bufs = (buf0, buf1)
    sems = (sem0, sem1)
    chunks = [(r, c) for r in range(ROWS_PER_W) for c in range(NCHUNK)]

    def start(g):
        r, c = chunks[g]
        return pltpu.async_copy(
            in_hbm.at[wid * ROWS_PER_W + r, pl.ds(c * CHUNK, CHUNK)],
            bufs[g & 1],
            sems[g & 1],
        )

    descs = {0: start(0)}
    mind = minv = None

    for g, (r, c) in enumerate(chunks):
        if g + 1 < len(chunks):
            descs[g + 1] = start(g + 1)
        descs.pop(g).wait()
        buf = bufs[g & 1]

        if c == 0:
            mind = [jnp.full((LANES,), _F32_BIG, jnp.float32)] * UNROLL
            minv = [jnp.zeros((LANES,), jnp.float32)] * UNROLL
        pv = pv_buf[pl.ds(r * LANES, LANES)]

        def body(i, carry, buf=buf, pv=pv):
            acc = list(carry)
            for k in range(UNROLL):
                v = buf[pl.ds(i * (LANES * UNROLL) + k * LANES, LANES)]
                d = jnp.abs(v - pv)
                md, mv = acc[k], acc[UNROLL + k]
                pred = d < md
                acc[k] = jnp.where(pred, d, md)
                acc[UNROLL + k] = jnp.where(pred, v, mv)
            return tuple(acc)

        res = lax.fori_loop(0, ITERS, body, tuple(mind) + tuple(minv))
        mind, minv = list(res[:UNROLL]), list(res[UNROLL:])

        if c == NCHUNK - 1:
            n = UNROLL
            while n > 1:
                n //= 2
                for k in range(n):
                    pred = mind[k + n] < mind[k]
                    mind[k] = jnp.where(pred, mind[k + n], mind[k])
                    minv[k] = jnp.where(pred, minv[k + n], minv[k])
            _, vs = plsc.sort_key_val(mind[0], minv[0])
            res_buf[...] = vs
            pltpu.sync_copy(res_buf, out_hbm.at[wid * ROWS_PER_W + r])


@jax.jit
def _closest(inp, prev_bc):
    mesh = plsc.VectorSubcoreMesh(core_axis_name="c", subcore_axis_name="s")
    f = pl.kernel(
        _closest_body,
        out_type=jax.ShapeDtypeStruct((BATCH, LANES), jnp.float32),
        mesh=mesh,
        compiler_params=pltpu.CompilerParams(needs_layout_passes=False),
        scratch_types=[
            pltpu.VMEM((CHUNK,), jnp.float32),
            pltpu.VMEM((CHUNK,), jnp.float32),
            pltpu.VMEM((ROWS_PER_W * LANES,), jnp.float32),
            pltpu.VMEM((LANES,), jnp.float32),
            pltpu.SemaphoreType.DMA,
            pltpu.SemaphoreType.DMA,
        ],
    )
    return f(inp, prev_bc)


def kernel(input, prev_output):
    prev_bc = jnp.broadcast_to(
        prev_output.reshape(NW, ROWS_PER_W, 1), (NW, ROWS_PER_W, LANES)
    ).reshape(NW, ROWS_PER_W * LANES)
    out = _closest(input, prev_bc)
    return out[:, :1]
